# Initial kernel scaffold; baseline (speedup 1.0000x reference)
import functools

import jax
import jax.numpy as jnp
from jax import lax
from jax.experimental import pallas as pl
from jax.experimental.pallas import tpu as pltpu
from jax.experimental.pallas import tpu_sc as plsc

_CHUNK = 128
_K_GROUP = 10


@functools.lru_cache(maxsize=None)
def _make_gather(n_rows: int, dim: int):
    info = plsc.get_sparse_core_info()
    nc, ns = info.num_cores, info.num_subcores
    nw = nc * ns
    assert n_rows % (nw * _CHUNK * _K_GROUP) == 0
    per_w = n_rows // nw
    chunks_w = per_w // _CHUNK
    n_groups = chunks_w // _K_GROUP
    group_rows = _K_GROUP * _CHUNK

    mesh = plsc.VectorSubcoreMesh(core_axis_name="c", subcore_axis_name="s")

    @functools.partial(
        pl.kernel,
        mesh=mesh,
        out_type=jax.ShapeDtypeStruct((n_rows, dim), jnp.float32),
        scratch_types=[
            pltpu.VMEM((chunks_w, _CHUNK), jnp.int32),
            pltpu.VMEM((group_rows, dim), jnp.float32),
            pltpu.SemaphoreType.DMA,
        ],
    )
    def gather_kernel(idx_hbm, table_hbm, out_hbm, idx_v, rows_v, sem):
        wid = lax.axis_index("s") * nc + lax.axis_index("c")
        chunk_base = wid * chunks_w
        row_base = wid * per_w
        pltpu.sync_copy(idx_hbm.at[pl.ds(chunk_base, chunks_w)], idx_v)

        def group_body(g, carry):
            copies = []
            for j in range(_K_GROUP):
                copies.append(pltpu.async_copy(
                    table_hbm.at[idx_v.at[g * _K_GROUP + j]],
                    rows_v.at[pl.ds(j * _CHUNK, _CHUNK)],
                    sem,
                ))
            for c in copies:
                c.wait()
            pltpu.sync_copy(
                rows_v, out_hbm.at[pl.ds(row_base + g * group_rows, group_rows)])
            return carry

        lax.fori_loop(0, n_groups, group_body, 0)

    return gather_kernel


def kernel(text_bt, aim_seq_len, table):
    b, s = text_bt.shape
    dim = table.shape[1]
    pos = lax.broadcasted_iota(jnp.int32, (1, s), 1)
    tb = jnp.where(pos < aim_seq_len, text_bt, 0)
    idx2d = tb.reshape(b * s // _CHUNK, _CHUNK)
    out = _make_gather(b * s, dim)(idx2d, table)
    return out.reshape(b, s, dim)


# speedup vs baseline: 3.2918x; 3.2918x over previous
import functools

import jax
import jax.numpy as jnp
from jax import lax
from jax.experimental import pallas as pl
from jax.experimental.pallas import tpu as pltpu
from jax.experimental.pallas import tpu_sc as plsc

_CHUNK = 128
_K_GROUP = 10


@functools.lru_cache(maxsize=None)
def _make_gather(n_rows: int, dim: int):
    info = plsc.get_sparse_core_info()
    nc, ns = info.num_cores, info.num_subcores
    nw = nc * ns
    assert n_rows % (nw * _CHUNK * _K_GROUP) == 0
    per_w = n_rows // nw
    chunks_w = per_w // _CHUNK
    n_groups = chunks_w // _K_GROUP
    group_rows = _K_GROUP * _CHUNK

    mesh = plsc.VectorSubcoreMesh(core_axis_name="c", subcore_axis_name="s")

    @functools.partial(
        pl.kernel,
        mesh=mesh,
        out_type=jax.ShapeDtypeStruct((n_rows, dim), jnp.float32),
        scratch_types=[
            pltpu.VMEM((chunks_w, _CHUNK), jnp.int32),
            pltpu.VMEM((group_rows, dim), jnp.float32),
            pltpu.SemaphoreType.DMA,
        ],
        compiler_params=pltpu.CompilerParams(use_tc_tiling_on_sc=False),
    )
    def gather_kernel(idx_hbm, table_hbm, out_hbm, idx_v, rows_v, sem):
        wid = lax.axis_index("s") * nc + lax.axis_index("c")
        row_base = wid * per_w
        pltpu.sync_copy(idx_hbm.at[wid], idx_v)

        def group_body(g, carry):
            copies = []
            for j in range(_K_GROUP):
                copies.append(pltpu.async_copy(
                    table_hbm.at[idx_v.at[g * _K_GROUP + j]],
                    rows_v.at[pl.ds(j * _CHUNK, _CHUNK)],
                    sem,
                ))
            for c in copies:
                c.wait()
            pltpu.sync_copy(
                rows_v, out_hbm.at[pl.ds(row_base + g * group_rows, group_rows)])
            return carry

        lax.fori_loop(0, n_groups, group_body, 0)

    return gather_kernel


def kernel(text_bt, aim_seq_len, table):
    b, s = text_bt.shape
    dim = table.shape[1]
    pos = lax.broadcasted_iota(jnp.int32, (1, s), 1)
    tb = jnp.where(pos < aim_seq_len, text_bt, 0)
    info = plsc.get_sparse_core_info()
    nw = info.num_cores * info.num_subcores
    idx3d = tb.reshape(nw, b * s // (nw * _CHUNK), _CHUNK)
    out = _make_gather(b * s, dim)(idx3d, table)
    return out.reshape(b, s, dim)
